# baseline (speedup 1.0000x reference)
import jax
import jax.numpy as jnp
from jax.experimental import pallas as pl


def kernel(h, edge_index, W, att_src, att_dst, bias):
    n = h.shape[0]
    x = h @ W
    a_src = (x * att_src).sum(-1)
    a_dst = (x * att_dst).sum(-1)
    loops = jnp.arange(n, dtype=edge_index.dtype)
    src = jnp.concatenate([edge_index[0], loops])
    dst = jnp.concatenate([edge_index[1], loops])
    alpha = a_src[src] + a_dst[dst]
    alpha = jax.nn.leaky_relu(alpha, negative_slope=0.2)
    amax = jax.ops.segment_max(alpha, dst, num_segments=n)
    alpha = jnp.exp(alpha - amax[dst])
    denom = jax.ops.segment_sum(alpha, dst, num_segments=n)
    alpha = alpha / denom[dst]
    msg = x[src] * alpha[:, None]
    out = jax.ops.segment_sum(msg, dst, num_segments=n) + bias
    scores = out.squeeze(-1)
    perm = jnp.argsort(-scores)
    perm_f = perm.astype(jnp.float32) + (scores - jax.lax.stop_gradient(scores))
    perm_idx = perm_f.astype(jnp.int32)
    h_ordered = h[perm_idx]
    return (h_ordered, perm_idx, scores)


# speedup vs baseline: 7.1785x; 7.1785x over previous
import functools

import jax
import jax.numpy as jnp
from jax import lax
from jax.experimental import pallas as pl
from jax.experimental.pallas import tpu as pltpu, tpu_sc as plsc

N = 10000
D = 256
E = 160000
E2 = E + N
NP = 10240
T = 10496
EP = 172032
PER_W = EP // 32
NVEC = PER_W // 16
SLICE = T // 16

_info = plsc.get_sparse_core_info()
NC, NS, L = _info.num_cores, _info.num_subcores, _info.num_lanes
NW = NC * NS
_mesh = plsc.VectorSubcoreMesh(core_axis_name="c", subcore_axis_name="s")

NEG_INF = float("-inf")


def _mv_kern(h_ref, w_ref, o_ref):
    o_ref[...] = lax.dot_general(
        h_ref[...], w_ref[...], (((1,), (0,)), ((), ())),
        preferred_element_type=jnp.float32)[:, 0]


def _matvec(h_pad, W):
    return pl.pallas_call(
        _mv_kern,
        grid=(NP // 1024,),
        in_specs=[pl.BlockSpec((1024, D), lambda i: (i, 0)),
                  pl.BlockSpec((D, 1), lambda i: (0, 0))],
        out_specs=pl.BlockSpec((1024,), lambda i: (i,)),
        out_shape=jax.ShapeDtypeStruct((NP,), jnp.float32),
    )(h_pad, W)


@functools.partial(
    pl.kernel, mesh=_mesh,
    compiler_params=pltpu.CompilerParams(needs_layout_passes=False),
    out_type=[jax.ShapeDtypeStruct((EP,), jnp.float32),
              jax.ShapeDtypeStruct((EP,), jnp.float32),
              jax.ShapeDtypeStruct((2 * T,), jnp.float32)],
    scratch_types=[
        pltpu.VMEM((T,), jnp.float32),
        pltpu.VMEM((T,), jnp.float32),
        pltpu.VMEM((PER_W,), jnp.int32),
        pltpu.VMEM((PER_W,), jnp.int32),
        pltpu.VMEM((PER_W,), jnp.float32),
        pltpu.VMEM((PER_W,), jnp.float32),
        pltpu.VMEM((16,), jnp.float32),
        pltpu.VMEM((16,), jnp.float32),
        pltpu.VMEM((SLICE,), jnp.float32),
        pltpu.VMEM((SLICE,), jnp.float32),
        pltpu.VMEM_SHARED((16 * T,), jnp.float32),
    ],
)
def _k2(x_hbm, src_hbm, dst_hbm, atts_hbm, attd_hbm,
        alpha_hbm, xs_hbm, pmax_hbm,
        xv, maxv, srcv, dstv, av, xsv, attsv, attdv, accv, stgv, shmax):
    cid = lax.axis_index("c")
    sid = lax.axis_index("s")
    wid = sid * NC + cid
    base = wid * PER_W

    pltpu.sync_copy(x_hbm, xv)
    pltpu.sync_copy(src_hbm.at[pl.ds(base, PER_W)], srcv)
    pltpu.sync_copy(dst_hbm.at[pl.ds(base, PER_W)], dstv)
    pltpu.sync_copy(atts_hbm, attsv)
    pltpu.sync_copy(attd_hbm, attdv)

    neg = jnp.full((L,), NEG_INF, jnp.float32)

    def init_body(i, _):
        off = pl.multiple_of(i * L, L)
        maxv[pl.ds(off, L)] = neg
        return 0

    lax.fori_loop(0, T // L, init_body, 0)

    att_s = attsv[...]
    att_d = attdv[...]

    def edge_body(i, _):
        off = pl.multiple_of(i * L, L)
        s16 = srcv[pl.ds(off, L)]
        d16 = dstv[pl.ds(off, L)]
        xs = plsc.load_gather(xv, [s16])
        xd = plsc.load_gather(xv, [d16])
        a = att_s * xs + att_d * xd
        alpha = jnp.where(a >= 0, a, a * jnp.float32(0.2))
        av[pl.ds(off, L)] = alpha
        xsv[pl.ds(off, L)] = xs

        cur = plsc.load_gather(maxv, [d16])

        def retry(r, m):
            plsc.store_scatter(maxv, [d16], alpha, mask=m)
            cur2 = plsc.load_gather(maxv, [d16])
            return m & (alpha > cur2)

        lax.fori_loop(0, L, retry, alpha > cur)
        return 0

    lax.fori_loop(0, NVEC, edge_body, 0)

    pltpu.sync_copy(av, alpha_hbm.at[pl.ds(base, PER_W)])
    pltpu.sync_copy(xsv, xs_hbm.at[pl.ds(base, PER_W)])

    pltpu.sync_copy(maxv, shmax.at[pl.ds(sid * T, T)])
    plsc.subcore_barrier()
    off = sid * SLICE
    pltpu.sync_copy(shmax.at[pl.ds(off, SLICE)], accv)

    def comb_body(s, _):
        pltpu.sync_copy(shmax.at[pl.ds(s * T + off, SLICE)], stgv)

        def vec_body(i, _):
            off2 = pl.multiple_of(i * L, L)
            accv[pl.ds(off2, L)] = jnp.maximum(
                accv[pl.ds(off2, L)], stgv[pl.ds(off2, L)])
            return 0

        lax.fori_loop(0, SLICE // L, vec_body, 0)
        return 0

    lax.fori_loop(1, 16, comb_body, 0)
    pltpu.sync_copy(accv, pmax_hbm.at[pl.ds(cid * T + off, SLICE)])


@functools.partial(
    pl.kernel, mesh=_mesh,
    compiler_params=pltpu.CompilerParams(needs_layout_passes=False),
    out_type=jax.ShapeDtypeStruct((EP,), jnp.float32),
    scratch_types=[
        pltpu.VMEM((T,), jnp.float32),
        pltpu.VMEM((T,), jnp.float32),
        pltpu.VMEM((PER_W,), jnp.float32),
        pltpu.VMEM((PER_W,), jnp.int32),
        pltpu.VMEM((PER_W,), jnp.float32),
    ],
)
def _k3(pmax_hbm, alpha_hbm, dst_hbm, exp_hbm, amaxv, othv, av, dstv, ev):
    cid = lax.axis_index("c")
    sid = lax.axis_index("s")
    wid = sid * NC + cid
    base = wid * PER_W

    pltpu.sync_copy(pmax_hbm.at[pl.ds(0, T)], amaxv)
    pltpu.sync_copy(pmax_hbm.at[pl.ds(T, T)], othv)

    def max_body(i, _):
        off = pl.multiple_of(i * L, L)
        amaxv[pl.ds(off, L)] = jnp.maximum(
            amaxv[pl.ds(off, L)], othv[pl.ds(off, L)])
        return 0

    lax.fori_loop(0, T // L, max_body, 0)

    pltpu.sync_copy(alpha_hbm.at[pl.ds(base, PER_W)], av)
    pltpu.sync_copy(dst_hbm.at[pl.ds(base, PER_W)], dstv)

    def edge_body(i, _):
        off = pl.multiple_of(i * L, L)
        d16 = dstv[pl.ds(off, L)]
        ag = plsc.load_gather(amaxv, [d16])
        ev[pl.ds(off, L)] = jnp.exp(av[pl.ds(off, L)] - ag)
        return 0

    lax.fori_loop(0, NVEC, edge_body, 0)
    pltpu.sync_copy(ev, exp_hbm.at[pl.ds(base, PER_W)])


@functools.partial(
    pl.kernel, mesh=_mesh,
    compiler_params=pltpu.CompilerParams(needs_layout_passes=False),
    out_type=jax.ShapeDtypeStruct((EP,), jnp.float32),
    scratch_types=[
        pltpu.VMEM((T,), jnp.float32),
        pltpu.VMEM((PER_W,), jnp.float32),
        pltpu.VMEM((PER_W,), jnp.float32),
        pltpu.VMEM((PER_W,), jnp.int32),
        pltpu.VMEM((PER_W,), jnp.float32),
    ],
)
def _k4(den_hbm, exp_hbm, xs_hbm, dst_hbm, msg_hbm, denv, ev, xsv, dstv, mv):
    cid = lax.axis_index("c")
    sid = lax.axis_index("s")
    wid = sid * NC + cid
    base = wid * PER_W

    pltpu.sync_copy(den_hbm, denv)
    pltpu.sync_copy(exp_hbm.at[pl.ds(base, PER_W)], ev)
    pltpu.sync_copy(xs_hbm.at[pl.ds(base, PER_W)], xsv)
    pltpu.sync_copy(dst_hbm.at[pl.ds(base, PER_W)], dstv)

    def edge_body(i, _):
        off = pl.multiple_of(i * L, L)
        d16 = dstv[pl.ds(off, L)]
        dg = plsc.load_gather(denv, [d16])
        aln = ev[pl.ds(off, L)] / dg
        mv[pl.ds(off, L)] = xsv[pl.ds(off, L)] * aln
        return 0

    lax.fori_loop(0, NVEC, edge_body, 0)
    pltpu.sync_copy(mv, msg_hbm.at[pl.ds(base, PER_W)])


IB = 256
NB = NP // IB


def _rank_kern(out2_ref, bias_ref, scores_ref, rank_ref, kref, accref):
    s = out2_ref[...] + bias_ref[...]
    scores_ref[...] = s
    t = lax.bitcast_convert_type(s, jnp.uint32)
    flip = jnp.where(t >= jnp.uint32(0x80000000),
                     jnp.uint32(0xFFFFFFFF), jnp.uint32(0x80000000))
    k = t ^ flip
    idx = lax.broadcasted_iota(jnp.int32, (NP,), 0)
    kref[...] = jnp.where(idx < N, k, jnp.uint32(0))

    def ib_body(ib, _):
        ui = kref[pl.ds(ib * IB, IB)].reshape(IB, 1)
        ii = lax.broadcasted_iota(jnp.int32, (IB, 1), 0) + ib * IB
        accref[...] = jnp.zeros((IB, IB), jnp.int32)

        def jb_body(jb, _):
            uj = kref[pl.ds(jb * IB, IB)].reshape(1, IB)
            jj = lax.broadcasted_iota(jnp.int32, (1, IB), 1) + jb * IB
            c = (uj > ui) | ((uj == ui) & (jj < ii))
            accref[...] = accref[...] + c.astype(jnp.int32)
            return 0

        lax.fori_loop(0, NB, jb_body, 0)
        rank_ref[pl.ds(ib * IB, IB)] = jnp.sum(accref[...], axis=1)
        return 0

    lax.fori_loop(0, NB, ib_body, 0)


def _rank(out2_pad, bias):
    return pl.pallas_call(
        _rank_kern,
        out_shape=[jax.ShapeDtypeStruct((NP,), jnp.float32),
                   jax.ShapeDtypeStruct((NP,), jnp.int32)],
        scratch_shapes=[pltpu.VMEM((NP,), jnp.uint32),
                        pltpu.VMEM((IB, IB), jnp.int32)],
    )(out2_pad, bias)


NCHUNK = 125


@functools.partial(
    pl.kernel, mesh=_mesh,
    compiler_params=pltpu.CompilerParams(needs_layout_passes=False),
    out_type=[jax.ShapeDtypeStruct((N, D), jnp.float32),
              jax.ShapeDtypeStruct((N,), jnp.int32)],
    scratch_types=[
        pltpu.VMEM((80,), jnp.int32),
        pltpu.VMEM((80, D), jnp.float32),
        pltpu.VMEM((80,), jnp.int32),
        pltpu.SemaphoreType.DMA,
        pltpu.SemaphoreType.DMA,
    ],
)
def _k6(rank_hbm, h_hbm, hord_hbm, perm_hbm, rkv, hbuf, iotv, sem1, sem2):
    cid = lax.axis_index("c")
    sid = lax.axis_index("s")
    wid = sid * NC + cid

    for kk in range(4):
        cidx = wid + 32 * kk

        @pl.when(cidx < NCHUNK)
        def _():
            rbase = cidx * 80
            pltpu.sync_copy(rank_hbm.at[pl.ds(rbase, 80)], rkv)
            pltpu.sync_copy(h_hbm.at[pl.ds(rbase, 80)], hbuf)

            def iota_body(j, _):
                off = pl.multiple_of(j * L, L)
                iotv[pl.ds(off, L)] = (
                    lax.broadcasted_iota(jnp.int32, (L,), 0)
                    + (rbase + j * L))
                return 0

            lax.fori_loop(0, 80 // L, iota_body, 0)
            pltpu.async_copy(hbuf, hord_hbm.at[rkv], sem1).wait()
            pltpu.async_copy(iotv, perm_hbm.at[rkv], sem2).wait()


def kernel(h, edge_index, W, att_src, att_dst, bias):
    loops = jnp.arange(N, dtype=edge_index.dtype)
    src = jnp.concatenate([edge_index[0], loops])
    dst = jnp.concatenate([edge_index[1], loops])
    pad_e = EP - E2
    src_all = jnp.concatenate([src, jnp.zeros((pad_e,), jnp.int32)])
    dst_all = jnp.concatenate(
        [dst, NP + (jnp.arange(pad_e, dtype=jnp.int32) % 16)])

    h_pad = jnp.pad(h, ((0, NP - N), (0, 0)))
    x = _matvec(h_pad, W)
    x_tab = jnp.pad(x, (0, T - NP))

    atts = jnp.full((16,), att_src[0], jnp.float32)
    attd = jnp.full((16,), att_dst[0], jnp.float32)

    alpha0, xs_g, pmax = _k2(x_tab, src_all, dst_all, atts, attd)
    expd = _k3(pmax, alpha0, dst_all)

    den = jax.ops.segment_sum(expd[:E2], dst, num_segments=N)
    den_tab = jnp.concatenate([den, jnp.ones((T - N,), jnp.float32)])

    msg = _k4(den_tab, expd, xs_g, dst_all)
    out2 = jax.ops.segment_sum(msg[:E2, None], dst, num_segments=N)[:, 0]

    out2_pad = jnp.pad(out2, (0, NP - N))
    bias_b = jnp.full((NP,), bias[0], jnp.float32)
    scores_full, rank = _rank(out2_pad, bias_b)

    h_ordered, perm = _k6(rank, h)
    return (h_ordered, perm, scores_full[:N])
